# baseline (speedup 1.0000x reference)
import functools

import jax
import jax.numpy as jnp
from jax import lax
from jax.experimental import pallas as pl
from jax.experimental.pallas import tpu as pltpu
from jax.experimental.pallas import tpu_sc as plsc

_ROWS = 16906
_TAB = 16907
_DIM = 200
_NW = 32
_CHUNK = 512
_BULK = _NW * _CHUNK
_TC_BLK = 1024
_TC_PAD = _TC_BLK

_mesh = plsc.ScalarSubcoreMesh(axis_name="c", num_cores=2)
_SCS_CHUNK = 4096


@functools.partial(
    pl.kernel,
    mesh=_mesh,
    out_type=jax.ShapeDtypeStruct((_DIM, _ROWS), jnp.float32),
    scratch_types=[
        pltpu.VMEM_SHARED((_DIM, _SCS_CHUNK), jnp.float32),
        pltpu.VMEM_SHARED((_DIM, _SCS_CHUNK), jnp.float32),
        pltpu.SemaphoreType.DMA,
        pltpu.SemaphoreType.DMA,
        pltpu.SemaphoreType.DMA,
        pltpu.SemaphoreType.DMA,
    ],
)
def _sc_bulk_copy(tab_hbm, out_hbm, buf_a, buf_b, sin_a, sin_b, sout_a,
                  sout_b):
    cid = lax.axis_index("c")
    base = cid * (2 * _SCS_CHUNK)
    in_a = pltpu.async_copy(tab_hbm.at[:, pl.ds(base, _SCS_CHUNK)], buf_a,
                            sin_a)
    in_b = pltpu.async_copy(tab_hbm.at[:, pl.ds(base + _SCS_CHUNK,
                                                _SCS_CHUNK)], buf_b, sin_b)
    in_a.wait()
    out_a = pltpu.async_copy(buf_a, out_hbm.at[:, pl.ds(base, _SCS_CHUNK)],
                             sout_a)
    in_b.wait()
    out_b = pltpu.async_copy(buf_b,
                             out_hbm.at[:, pl.ds(base + _SCS_CHUNK,
                                                 _SCS_CHUNK)], sout_b)
    out_a.wait()
    out_b.wait()


def _tc_tail_body(acc_ref, tab_ref, out_ref):
    del acc_ref
    out_ref[...] = tab_ref[...]


_tc_tail = pl.pallas_call(
    _tc_tail_body,
    grid=(1,),
    in_specs=[
        pl.BlockSpec(memory_space=pltpu.MemorySpace.HBM),
        pl.BlockSpec((_DIM, _TC_BLK), lambda j: (0, 0)),
    ],
    out_specs=pl.BlockSpec((_DIM, _TC_BLK), lambda j: (0, _BULK // _TC_BLK)),
    out_shape=jax.ShapeDtypeStruct((_DIM, _ROWS), jnp.float32),
    input_output_aliases={0: 0},
)


def kernel(x, table):
    del x
    tab_t = table.T
    bulk = _sc_bulk_copy(tab_t)
    tail = lax.slice(tab_t, (0, _BULK), (_DIM, _TAB))
    tail = jnp.pad(tail, ((0, 0), (0, _TC_PAD - (_TAB - _BULK))))
    out_t = _tc_tail(bulk, tail)
    return out_t.T


# speedup vs baseline: 1.0921x; 1.0921x over previous
import functools

import jax
import jax.numpy as jnp
from jax import lax
from jax.experimental import pallas as pl
from jax.experimental.pallas import tpu as pltpu
from jax.experimental.pallas import tpu_sc as plsc

_ROWS = 16906
_TAB = 16907
_DIM = 200
_NW = 32
_CHUNK = 512
_BULK = _NW * _CHUNK
_TC_BLK = 1024
_TC_PAD = _TC_BLK

_mesh = plsc.VectorSubcoreMesh(core_axis_name="c", subcore_axis_name="s")


@functools.partial(
    pl.kernel,
    mesh=_mesh,
    out_type=jax.ShapeDtypeStruct((_DIM, _ROWS), jnp.float32),
    scratch_types=[pltpu.VMEM((_DIM, _CHUNK), jnp.float32)],
)
def _sc_bulk_copy(tab_hbm, out_hbm, buf):
    wid = lax.axis_index("s") * 2 + lax.axis_index("c")
    base = wid * _CHUNK
    pltpu.sync_copy(tab_hbm.at[:, pl.ds(base, _CHUNK)], buf)
    pltpu.sync_copy(buf, out_hbm.at[:, pl.ds(base, _CHUNK)])


def _tc_tail_body(acc_ref, tab_ref, out_ref):
    del acc_ref
    out_ref[...] = tab_ref[...]


_tc_tail = pl.pallas_call(
    _tc_tail_body,
    grid=(1,),
    in_specs=[
        pl.BlockSpec(memory_space=pltpu.MemorySpace.HBM),
        pl.BlockSpec((_DIM, _TC_BLK), lambda j: (0, 0)),
    ],
    out_specs=pl.BlockSpec((_DIM, _TC_BLK), lambda j: (0, _BULK // _TC_BLK)),
    out_shape=jax.ShapeDtypeStruct((_DIM, _ROWS), jnp.float32),
    input_output_aliases={0: 0},
)


def kernel(x, table):
    del x
    tab_t = table.T
    bulk = _sc_bulk_copy(tab_t)
    tail = lax.slice(tab_t, (0, _BULK), (_DIM, _TAB))
    tail = jnp.pad(tail, ((0, 0), (0, _TC_PAD - (_TAB - _BULK))))
    out_t = _tc_tail(bulk, tail)
    return out_t.T
